# baseline (speedup 1.0000x reference)
import functools

import jax
import jax.numpy as jnp
from jax import lax
from jax.experimental import pallas as pl
from jax.experimental.pallas import tpu as pltpu
from jax.experimental.pallas import tpu_sc as plsc

PAGE = 64
SEQS = 64
PAGES = 4096
PPS = 128
D = 128
TOKENS = 2048
ROWS_PER_BLK = 8192
N_BLK = (PAGES * PAGE) // ROWS_PER_BLK
L = 16


def _sc_meta(counts_hbm, tokens_hbm, upd_hbm,
             pi_hbm, bpi_hbm, po_hbm, nl_hbm, bsl_hbm, cu_hbm, ns_hbm,
             dests_hbm,
             c_v, tok_v, u_v, starts_v, npg_v, val_v, arr_v,
             po_v, row_v, d_v, nl_v, cu_v, ns_v):
    wid = lax.axis_index("s") * 2 + lax.axis_index("c")
    pltpu.sync_copy(counts_hbm, c_v)
    pltpu.sync_copy(tokens_hbm.at[pl.ds(64 * wid, 64)], tok_v)

    iota = lax.iota(jnp.int32, L)
    carry_pg = jnp.int32(0)
    carry_tok = jnp.int32(0)
    for k in range(4):
        sl = pl.ds(L * k, L)
        c = c_v[sl]
        npg = (c + PAGE - 1) // PAGE
        ends = plsc.cumsum(npg) + carry_pg
        starts = ends - npg
        cui = plsc.cumsum(c) + carry_tok
        cuex = cui - c
        starts_v[sl] = starts
        npg_v[sl] = npg
        val_v[sl] = PAGE * starts - cuex
        nl_v[sl] = jnp.where(c > 0, c, -1)
        cu_v[sl] = cuex
        carry_pg = carry_pg + jnp.sum(npg)
        carry_tok = carry_tok + jnp.sum(c)
    total_pages = carry_pg
    cu_v[pl.ds(64, L)] = jnp.where(iota == 0, carry_tok, 0)

    for k in range(4):
        t = tok_v[pl.ds(L * k, L)]
        g = plsc.load_gather(val_v, [t])
        d_v[pl.ds(L * k, L)] = g + (64 * wid + L * k) + iota
    pltpu.sync_copy(d_v, dests_hbm.at[pl.ds(64 * wid, 64)])

    for r in range(2):
        sidx = jnp.full((L,), 2 * wid + r, jnp.int32)
        st = plsc.load_gather(starts_v, [sidx])
        npb = plsc.load_gather(npg_v, [sidx])
        for k in range(8):
            j = iota + L * k
            row_v[r, pl.ds(L * k, L)] = jnp.where(j < npb, st + j, -1)
    pltpu.sync_copy(row_v, pi_hbm.at[pl.ds(2 * wid, 2)])
    pltpu.sync_copy(row_v, bpi_hbm.at[pl.ds(2 * wid, 2)])

    @pl.when(wid != 0)
    def _():
        for k in range(8):
            po_v[pl.ds(L * k, L)] = jnp.full((L,), -1, jnp.int32)

    @pl.when(wid == 0)
    def _():
        for k in range(8):
            arr_v[pl.ds(L * k, L)] = jnp.zeros((L,), jnp.int32)
        for k in range(4):
            st = starts_v[pl.ds(L * k, L)]
            npg = npg_v[pl.ds(L * k, L)]
            plsc.store_scatter(arr_v, [st], iota + L * k + 1, mask=npg > 0)
        carry = jnp.int32(0)
        for k in range(8):
            cm = jnp.maximum(plsc.cummax(arr_v[pl.ds(L * k, L)]), carry)
            p = iota + L * k
            po_v[pl.ds(L * k, L)] = jnp.where(p < total_pages, cm - 1, -1)
            carry = jnp.max(cm)

    pltpu.sync_copy(po_v, po_hbm.at[pl.ds(128 * wid, 128)])

    @pl.when(wid == 1)
    def _():
        pltpu.sync_copy(nl_v, nl_hbm)
        pltpu.sync_copy(nl_v, bsl_hbm)

    @pl.when(wid == 2)
    def _():
        pltpu.sync_copy(cu_v.at[pl.ds(0, SEQS + 1)], cu_hbm)

    @pl.when(wid == 3)
    def _():
        pltpu.sync_copy(upd_hbm, u_v)
        nsum = jnp.int32(0)
        for k in range(4):
            u = u_v[pl.ds(L * k, L)]
            nsum = nsum + jnp.sum(jnp.where(u >= 0, 1, 0))
        ns_v[...] = jnp.where(iota == 0, nsum, 0)
        pltpu.sync_copy(ns_v.at[pl.ds(0, 1)], ns_hbm)


DATA_PAGES = 96


def _cache_kernel(c_smem, kv_ref, out_ref, kvp_v, srcb_s, vcnt_s):
    b = pl.program_id(0)

    @pl.when(b == 0)
    def _():
        kvp_v[0:TOKENS, :] = kv_ref[...]
        kvp_v[TOKENS:, :] = jnp.zeros((PAGE, D), jnp.float32)

        def init_body(p, _):
            srcb_s[p] = 0
            vcnt_s[p] = 0
            return 0

        lax.fori_loop(0, PPS, init_body, 0)

        def seq_body(s, carry):
            page, tok = carry
            cs = c_smem[0, s]
            npgs = (cs + PAGE - 1) // PAGE

            def page_body(j, _):
                srcb_s[page + j] = tok + PAGE * j
                vcnt_s[page + j] = cs - PAGE * j
                return 0

            lax.fori_loop(0, npgs, page_body, 0)
            return (page + npgs, tok + cs)

        lax.fori_loop(0, SEQS, seq_body, (jnp.int32(0), jnp.int32(0)))

    @pl.when(b != N_BLK - 1)
    def _():
        out_ref[...] = jnp.zeros_like(out_ref)

    @pl.when(b == N_BLK - 1)
    def _():
        rid = lax.broadcasted_iota(jnp.int32, (PAGE, 1), 0)
        for p in range(DATA_PAGES):
            sb = srcb_s[p]
            vc = vcnt_s[p]
            rows = kvp_v[pl.ds(sb, PAGE), :]
            out_ref[PAGE * p:PAGE * (p + 1), :] = jnp.where(rid < vc, rows, 0.0)
        out_ref[PAGE * DATA_PAGES:, :] = jnp.zeros(
            (ROWS_PER_BLK - PAGE * DATA_PAGES, D), jnp.float32)


_SC_MESH = plsc.VectorSubcoreMesh(core_axis_name="c", subcore_axis_name="s")
i32 = jnp.int32

_sc_meta_call = functools.partial(
    pl.kernel, mesh=_SC_MESH,
    compiler_params=pltpu.CompilerParams(needs_layout_passes=False),
    out_type=[
        jax.ShapeDtypeStruct((SEQS, PPS), i32),
        jax.ShapeDtypeStruct((SEQS, PPS), i32),
        jax.ShapeDtypeStruct((PAGES,), i32),
        jax.ShapeDtypeStruct((SEQS,), i32),
        jax.ShapeDtypeStruct((SEQS,), i32),
        jax.ShapeDtypeStruct((SEQS + 1,), i32),
        jax.ShapeDtypeStruct((1,), i32),
        jax.ShapeDtypeStruct((TOKENS,), i32),
    ],
    scratch_types=[
        pltpu.VMEM((64,), i32),
        pltpu.VMEM((64,), i32),
        pltpu.VMEM((64,), i32),
        pltpu.VMEM((64,), i32),
        pltpu.VMEM((64,), i32),
        pltpu.VMEM((64,), i32),
        pltpu.VMEM((128,), i32),
        pltpu.VMEM((128,), i32),
        pltpu.VMEM((2, 128), i32),
        pltpu.VMEM((64,), i32),
        pltpu.VMEM((64,), i32),
        pltpu.VMEM((128,), i32),
        pltpu.VMEM((L,), i32),
    ],
)(_sc_meta)


@jax.jit
def kernel(kv_cache, new_kv, updated_seqs, new_counts, tokens,
           page_indices, page_owners, seq_lens):
    del kv_cache, page_indices, page_owners, seq_lens

    new_cache = pl.pallas_call(
        _cache_kernel,
        grid=(N_BLK,),
        in_specs=[
            pl.BlockSpec(memory_space=pltpu.SMEM),
            pl.BlockSpec((TOKENS, D), lambda b: (0, 0)),
        ],
        out_specs=pl.BlockSpec((ROWS_PER_BLK, D), lambda b: ((b + 1) % N_BLK, 0)),
        out_shape=jax.ShapeDtypeStruct((PAGES * PAGE, D), jnp.float32),
        scratch_shapes=[
            pltpu.VMEM((TOKENS + PAGE, D), jnp.float32),
            pltpu.SMEM((PPS,), jnp.int32),
            pltpu.SMEM((PPS,), jnp.int32),
        ],
    )(new_counts.reshape(1, SEQS), new_kv)

    pi, bpi, po, nl, bsl, cu, ns1, dests = _sc_meta_call(
        new_counts, tokens, updated_seqs)

    ns = ns1.reshape(())
    return (new_cache, pi, po, nl, bpi, bsl, cu, ns, dests)
